# Initial kernel scaffold; baseline (speedup 1.0000x reference)
import jax
import jax.numpy as jnp
from jax.experimental import pallas as pl


def kernel(x, Wk, bk, Wq, bq, Wv, bv, Wp, bp, Wg1, bg1, Wg2, bg2, g1, beta1, g2, beta2):
    raise NotImplementedError("write your pallas kernel here")


# speedup vs baseline: 43.9880x; 43.9880x over previous
import functools

import jax
import jax.numpy as jnp
from jax import lax
from jax.experimental import pallas as pl

PLANE = 96
INTER = 96
HEADS = 4
OUTP = 96
TOPK = 5
F = INTER * HEADS
N = 1024
B = 8
EPS = 1e-5
CNT = float(B * N)

_f32 = jnp.float32


def _dot(a, b, dims):
    return lax.dot_general(a, b, (dims, ((), ())),
                           preferred_element_type=_f32)


def _headnorm(t):
    outs = []
    for h in range(HEADS):
        ch = t[h * INTER:(h + 1) * INTER, :]
        ss = jnp.sum(ch * ch, axis=0, keepdims=True)
        outs.append(ch / jnp.maximum(jnp.sqrt(ss), 1e-12))
    return jnp.concatenate(outs, axis=0)


def _k1(x_ref, wk_ref, bk_ref, wq_ref, bq_ref, wv_ref, bv_ref,
        wp_ref, bp_ref, wg1_ref, bg1_ref, z1_ref, st1_ref):
    b = pl.program_id(0)
    xf = x_ref[0]

    k = _dot(wk_ref[...], xf, ((1,), (0,))) + bk_ref[...]
    q = _dot(wq_ref[...], xf, ((1,), (0,))) + bq_ref[...]
    v = _dot(wv_ref[...], xf, ((1,), (0,))) + bv_ref[...]
    k = _headnorm(k)
    q = _headnorm(q)

    ssx = jnp.sum(xf * xf, axis=0, keepdims=True)
    xn = xf / jnp.maximum(jnp.sqrt(ssx), 1e-12)
    sim = _dot(xn, xn, ((0,), (0,)))

    coli = lax.broadcasted_iota(jnp.int32, (N, N), 1)
    rowi = lax.broadcasted_iota(jnp.int32, (N, N), 0)
    adj = jnp.where(rowi == coli, 1.0, 0.0).astype(_f32)
    for _ in range(TOPK):
        m = jnp.max(sim, axis=1, keepdims=True)
        cand = jnp.where(sim == m, coli, N)
        am = jnp.min(cand, axis=1, keepdims=True)
        hit = coli == am
        adj = adj + jnp.where(hit, 1.0, 0.0).astype(_f32)
        sim = jnp.where(hit, -jnp.inf, sim)

    lapk = _dot(k, adj, ((1,), (1,)))
    lapv = _dot(v, adj, ((1,), (1,)))
    kv = jnp.sum(lapk * lapv, axis=1, keepdims=True) * (1.0 / 36.0)
    hydra = q * kv

    y1 = _dot(wp_ref[...], hydra, ((1,), (0,))) + bp_ref[...]
    z1 = _dot(wg1_ref[...], y1, ((1,), (0,))) + bg1_ref[...]
    z1_ref[0] = z1

    @pl.when(b == 0)
    def _():
        st1_ref[...] = jnp.zeros_like(st1_ref)

    st1_ref[:, 0:1] = st1_ref[:, 0:1] + jnp.sum(z1, axis=1, keepdims=True)
    st1_ref[:, 1:2] = st1_ref[:, 1:2] + jnp.sum(z1 * z1, axis=1,
                                                keepdims=True)


def _k2(z1_ref, st1_ref, wg2_ref, bg2_ref, g1_ref, beta1_ref,
        z2_ref, st2_ref):
    b = pl.program_id(0)
    z = z1_ref[0]
    mean = st1_ref[:, 0:1] / CNT
    var = st1_ref[:, 1:2] / CNT - mean * mean
    y = (z - mean) * lax.rsqrt(var + EPS) * g1_ref[...] + beta1_ref[...]
    y = jnp.maximum(y, 0.0)
    z2 = _dot(wg2_ref[...], y, ((1,), (0,))) + bg2_ref[...]
    z2_ref[0] = z2

    @pl.when(b == 0)
    def _():
        st2_ref[...] = jnp.zeros_like(st2_ref)

    st2_ref[:, 0:1] = st2_ref[:, 0:1] + jnp.sum(z2, axis=1, keepdims=True)
    st2_ref[:, 1:2] = st2_ref[:, 1:2] + jnp.sum(z2 * z2, axis=1,
                                                keepdims=True)


def _k3(z2_ref, st2_ref, g2_ref, beta2_ref, out_ref):
    z = z2_ref[0]
    mean = st2_ref[:, 0:1] / CNT
    var = st2_ref[:, 1:2] / CNT - mean * mean
    y = (z - mean) * lax.rsqrt(var + EPS) * g2_ref[...] + beta2_ref[...]
    out_ref[0] = jnp.maximum(y, 0.0)


def _full(shape):
    return pl.BlockSpec(shape, lambda b: (0,) * len(shape))


def _batched(shape):
    return pl.BlockSpec(shape, lambda b: (b, 0, 0))


@jax.jit
def kernel(x, Wk, bk, Wq, bq, Wv, bv, Wp, bp, Wg1, bg1, Wg2, bg2,
           g1, beta1, g2, beta2):
    b, c, h, w = x.shape
    xr = x.reshape(b, c, h * w)
    col = lambda a: a.reshape(-1, 1)

    z1, st1 = pl.pallas_call(
        _k1,
        grid=(B,),
        in_specs=[_batched((1, PLANE, N)),
                  _full((F, PLANE)), _full((F, 1)),
                  _full((F, PLANE)), _full((F, 1)),
                  _full((F, PLANE)), _full((F, 1)),
                  _full((OUTP, F)), _full((OUTP, 1)),
                  _full((OUTP, OUTP)), _full((OUTP, 1))],
        out_specs=[_batched((1, OUTP, N)), _full((OUTP, 128))],
        out_shape=[jax.ShapeDtypeStruct((B, OUTP, N), _f32),
                   jax.ShapeDtypeStruct((OUTP, 128), _f32)],
    )(xr, Wk, col(bk), Wq, col(bq), Wv, col(bv), Wp, col(bp),
      Wg1, col(bg1))

    z2, st2 = pl.pallas_call(
        _k2,
        grid=(B,),
        in_specs=[_batched((1, OUTP, N)), _full((OUTP, 128)),
                  _full((OUTP, OUTP)), _full((OUTP, 1)),
                  _full((OUTP, 1)), _full((OUTP, 1))],
        out_specs=[_batched((1, OUTP, N)), _full((OUTP, 128))],
        out_shape=[jax.ShapeDtypeStruct((B, OUTP, N), _f32),
                   jax.ShapeDtypeStruct((OUTP, 128), _f32)],
    )(z1, st1, Wg2, col(bg2), col(g1), col(beta1))

    out = pl.pallas_call(
        _k3,
        grid=(B,),
        in_specs=[_batched((1, OUTP, N)), _full((OUTP, 128)),
                  _full((OUTP, 1)), _full((OUTP, 1))],
        out_specs=_batched((1, OUTP, N)),
        out_shape=jax.ShapeDtypeStruct((B, OUTP, N), _f32),
    )(z2, st2, col(g2), col(beta2))

    return out.reshape(b, OUTP, h, w)


# speedup vs baseline: 63.8516x; 1.4516x over previous
import functools

import jax
import jax.numpy as jnp
from jax import lax
from jax.experimental import pallas as pl

PLANE = 96
INTER = 96
HEADS = 4
OUTP = 96
TOPK = 5
F = INTER * HEADS
N = 1024
B = 8
EPS = 1e-5
CNT = float(B * N)

_f32 = jnp.float32


def _dot(a, b, dims):
    return lax.dot_general(a, b, (dims, ((), ())),
                           preferred_element_type=_f32)


def _headnorm(t):
    outs = []
    for h in range(HEADS):
        ch = t[h * INTER:(h + 1) * INTER, :]
        ss = jnp.sum(ch * ch, axis=0, keepdims=True)
        outs.append(ch / jnp.maximum(jnp.sqrt(ss), 1e-12))
    return jnp.concatenate(outs, axis=0)


def _k1(x_ref, wk_ref, bk_ref, wq_ref, bq_ref, wv_ref, bv_ref,
        wp_ref, bp_ref, wg1_ref, bg1_ref, z1_ref, st1_ref):
    b = pl.program_id(0)
    xf = x_ref[0]

    k = _dot(wk_ref[...], xf, ((1,), (0,))) + bk_ref[...]
    q = _dot(wq_ref[...], xf, ((1,), (0,))) + bq_ref[...]
    v = _dot(wv_ref[...], xf, ((1,), (0,))) + bv_ref[...]
    k = _headnorm(k)
    q = _headnorm(q)

    ssx = jnp.sum(xf * xf, axis=0, keepdims=True)
    xn = xf / jnp.maximum(jnp.sqrt(ssx), 1e-12)
    sim = _dot(xn, xn, ((0,), (0,)))

    coli = lax.broadcasted_iota(jnp.int32, (N, N), 1)
    rowi = lax.broadcasted_iota(jnp.int32, (N, N), 0)
    eye = jnp.where(rowi == coli, 1.0, 0.0).astype(_f32)
    s = sim
    for _ in range(TOPK - 1):
        m = jnp.max(s, axis=1, keepdims=True)
        s = jnp.where(s == m, -jnp.inf, s)
    t5 = jnp.max(s, axis=1, keepdims=True)
    adj = jnp.where(sim >= t5, 1.0, 0.0).astype(_f32) + eye

    lapk = _dot(k, adj, ((1,), (1,)))
    lapv = _dot(v, adj, ((1,), (1,)))
    kv = jnp.sum(lapk * lapv, axis=1, keepdims=True) * (1.0 / 36.0)
    hydra = q * kv

    y1 = _dot(wp_ref[...], hydra, ((1,), (0,))) + bp_ref[...]
    z1 = _dot(wg1_ref[...], y1, ((1,), (0,))) + bg1_ref[...]
    z1_ref[0] = z1

    @pl.when(b == 0)
    def _():
        st1_ref[...] = jnp.zeros_like(st1_ref)

    st1_ref[:, 0:1] = st1_ref[:, 0:1] + jnp.sum(z1, axis=1, keepdims=True)
    st1_ref[:, 1:2] = st1_ref[:, 1:2] + jnp.sum(z1 * z1, axis=1,
                                                keepdims=True)


def _k2(z1_ref, st1_ref, wg2_ref, bg2_ref, g1_ref, beta1_ref,
        z2_ref, st2_ref):
    b = pl.program_id(0)
    z = z1_ref[0]
    mean = st1_ref[:, 0:1] / CNT
    var = st1_ref[:, 1:2] / CNT - mean * mean
    y = (z - mean) * lax.rsqrt(var + EPS) * g1_ref[...] + beta1_ref[...]
    y = jnp.maximum(y, 0.0)
    z2 = _dot(wg2_ref[...], y, ((1,), (0,))) + bg2_ref[...]
    z2_ref[0] = z2

    @pl.when(b == 0)
    def _():
        st2_ref[...] = jnp.zeros_like(st2_ref)

    st2_ref[:, 0:1] = st2_ref[:, 0:1] + jnp.sum(z2, axis=1, keepdims=True)
    st2_ref[:, 1:2] = st2_ref[:, 1:2] + jnp.sum(z2 * z2, axis=1,
                                                keepdims=True)


def _k3(z2_ref, st2_ref, g2_ref, beta2_ref, out_ref):
    z = z2_ref[0]
    mean = st2_ref[:, 0:1] / CNT
    var = st2_ref[:, 1:2] / CNT - mean * mean
    y = (z - mean) * lax.rsqrt(var + EPS) * g2_ref[...] + beta2_ref[...]
    out_ref[0] = jnp.maximum(y, 0.0)


def _full(shape):
    return pl.BlockSpec(shape, lambda b: (0,) * len(shape))


def _batched(shape):
    return pl.BlockSpec(shape, lambda b: (b, 0, 0))


@jax.jit
def kernel(x, Wk, bk, Wq, bq, Wv, bv, Wp, bp, Wg1, bg1, Wg2, bg2,
           g1, beta1, g2, beta2):
    b, c, h, w = x.shape
    xr = x.reshape(b, c, h * w)
    col = lambda a: a.reshape(-1, 1)

    z1, st1 = pl.pallas_call(
        _k1,
        grid=(B,),
        in_specs=[_batched((1, PLANE, N)),
                  _full((F, PLANE)), _full((F, 1)),
                  _full((F, PLANE)), _full((F, 1)),
                  _full((F, PLANE)), _full((F, 1)),
                  _full((OUTP, F)), _full((OUTP, 1)),
                  _full((OUTP, OUTP)), _full((OUTP, 1))],
        out_specs=[_batched((1, OUTP, N)), _full((OUTP, 128))],
        out_shape=[jax.ShapeDtypeStruct((B, OUTP, N), _f32),
                   jax.ShapeDtypeStruct((OUTP, 128), _f32)],
    )(xr, Wk, col(bk), Wq, col(bq), Wv, col(bv), Wp, col(bp),
      Wg1, col(bg1))

    z2, st2 = pl.pallas_call(
        _k2,
        grid=(B,),
        in_specs=[_batched((1, OUTP, N)), _full((OUTP, 128)),
                  _full((OUTP, OUTP)), _full((OUTP, 1)),
                  _full((OUTP, 1)), _full((OUTP, 1))],
        out_specs=[_batched((1, OUTP, N)), _full((OUTP, 128))],
        out_shape=[jax.ShapeDtypeStruct((B, OUTP, N), _f32),
                   jax.ShapeDtypeStruct((OUTP, 128), _f32)],
    )(z1, st1, Wg2, col(bg2), col(g1), col(beta1))

    out = pl.pallas_call(
        _k3,
        grid=(B,),
        in_specs=[_batched((1, OUTP, N)), _full((OUTP, 128)),
                  _full((OUTP, 1)), _full((OUTP, 1))],
        out_specs=_batched((1, OUTP, N)),
        out_shape=jax.ShapeDtypeStruct((B, OUTP, N), _f32),
    )(z2, st2, col(g2), col(beta2))

    return out.reshape(b, OUTP, h, w)
